# baseline (speedup 1.0000x reference)
import functools

import jax
import jax.numpy as jnp
from jax import lax
from jax.experimental import pallas as pl
from jax.experimental.pallas import tpu as pltpu

I_SIZE = 3
H = 64
G = 8 * H
SLOT = 2 * H
OUTPAD = 128


def _gate_cols(w, layer):
    i, f, g, o = jnp.split(w, 4, axis=1)
    z = jnp.zeros_like(i)
    pairs = ((i, z), (f, z), (o, z), (g, z)) if layer == 0 else \
            ((z, i), (z, f), (z, o), (z, g))
    return jnp.concatenate([blk for pair in pairs for blk in pair], axis=1)


def _lstm_kernel(xr_ref, p_ref, ones_ref, w_ref, wfcb_ref, bfct_ref,
                 out_ref, xu, hseq, yacc, *, T, TB):
    NS = T + 1
    xr = xr_ref[...]

    def _write_xu(lo, hi):
        xu[:, lo * SLOT:hi * SLOT] = (
            jnp.dot(xr, p_ref[:, lo * SLOT:hi * SLOT],
                    preferred_element_type=jnp.float32)
            + ones_ref[:, lo * SLOT:hi * SLOT]).astype(jnp.bfloat16)

    w = w_ref[...]
    lane = lax.broadcasted_iota(jnp.int32, (TB, 2 * H), 1)
    l0_mask = lane < H

    h = jnp.zeros((TB, 2 * H), jnp.bfloat16)
    c = jnp.zeros((TB, 2 * H), jnp.float32)

    def _fc_chunk(k0, k1, first):
        part = jnp.dot(hseq[:, k0 * 2 * H:k1 * 2 * H],
                       wfcb_ref[k0 * 2 * H:k1 * 2 * H, :],
                       preferred_element_type=jnp.float32)
        yacc[...] = part if first else yacc[...] + part

    CH = 8
    fc_chunks = [(k, min(k + CH, T)) for k in range(0, T, CH)]
    fc_at = {k1 + 1: i for i, (k0, k1) in enumerate(fc_chunks) if k1 + 1 < NS}

    _write_xu(0, min(CH, NS))
    for s in range(NS):
        if s % CH == 1 and s + CH - 1 < NS:
            _write_xu(s + CH - 1, min(s + 2 * CH - 1, NS))
        if s in fc_at:
            k0, k1 = fc_chunks[fc_at[s]]
            _fc_chunk(k0, k1, first=(fc_at[s] == 0))
        hx = jnp.concatenate([h, xu[:, s * SLOT:(s + 1) * SLOT]], axis=1)
        z = jnp.dot(hx, w, preferred_element_type=jnp.float32)
        t = jnp.tanh(z)
        ti = t[:, :2 * H]
        tf = t[:, 2 * H:4 * H]
        to = t[:, 4 * H:6 * H]
        tg = t[:, 6 * H:]
        c = 0.5 * ((tf * c + c) + (ti * tg + tg))
        tc = jnp.tanh(c)
        hf = 0.5 * (to * tc + tc)
        if s == 0:
            hf = jnp.where(l0_mask, hf, 0.0)
            c = jnp.where(l0_mask, c, 0.0)
            h = hf.astype(jnp.bfloat16)
        else:
            h = hf.astype(jnp.bfloat16)
            hseq[:, (s - 1) * 2 * H:s * 2 * H] = h

    done = set(fc_at.values())
    for i, (k0, k1) in enumerate(fc_chunks):
        if i not in done:
            _fc_chunk(k0, k1, first=(i == 0))
    y = yacc[...] + bfct_ref[...]
    out_ref[...] = y[:, :T * I_SIZE].astype(out_ref.dtype)


@jax.jit
def _forward(x, wih0, whh0, b0, wih1, whh1, b1, wfc, bfc):
    B, T, I = x.shape
    TB = 1024
    if B < TB:
        TB = max(8, -(-B // 8) * 8)
    Bpad = -(-B // TB) * TB
    nb = Bpad // TB
    xr = x.astype(jnp.bfloat16).reshape(B, T * I)
    if Bpad != B:
        xr = jnp.pad(xr, ((0, Bpad - B), (0, 0)))

    bb = _gate_cols(b0, 0) + _gate_cols(b1, 1)
    wx = _gate_cols(wih0, 0)
    wh = jnp.concatenate(
        [_gate_cols(whh0, 0) + _gate_cols(wih1, 1),
         _gate_cols(whh1, 1)], axis=0)

    wxe = jnp.zeros((SLOT, G), jnp.float32)
    wxe = wxe.at[:I, :].set(wx).at[I, :].set(bb[0])
    w = jnp.concatenate([wh, wxe], axis=0)
    w = w.at[:, :6 * H].multiply(0.5).astype(jnp.bfloat16)

    NS = T + 1
    p = jnp.zeros((T * I, NS * SLOT), jnp.float32)
    for t in range(T):
        p = p.at[t * I:(t + 1) * I, t * SLOT:t * SLOT + I].set(jnp.eye(I))
    p = p.astype(jnp.bfloat16)
    ones_row = jnp.zeros((1, NS * SLOT), jnp.float32)
    ones_row = ones_row.at[0, jnp.arange(NS) * SLOT + I].set(1.0)

    wfcb = jnp.zeros((T * 2 * H, OUTPAD), jnp.float32)
    for t in range(T):
        wfcb = wfcb.at[t * 2 * H + H:(t + 1) * 2 * H, t * I:(t + 1) * I].set(wfc)
    wfcb = wfcb.astype(jnp.bfloat16)
    bfct = jnp.zeros((1, OUTPAD), jnp.float32).at[:, :T * I].set(
        jnp.tile(bfc, (1, T)))

    out = pl.pallas_call(
        functools.partial(_lstm_kernel, T=T, TB=TB),
        out_shape=jax.ShapeDtypeStruct((Bpad, T * I), jnp.float32),
        grid=(nb,),
        in_specs=[
            pl.BlockSpec((TB, T * I), lambda i: (i, 0)),
            pl.BlockSpec((T * I, NS * SLOT), lambda i: (0, 0)),
            pl.BlockSpec((1, NS * SLOT), lambda i: (0, 0)),
            pl.BlockSpec((2 * H + SLOT, G), lambda i: (0, 0)),
            pl.BlockSpec((T * 2 * H, OUTPAD), lambda i: (0, 0)),
            pl.BlockSpec((1, OUTPAD), lambda i: (0, 0)),
        ],
        out_specs=pl.BlockSpec((TB, T * I), lambda i: (i, 0)),
        scratch_shapes=[
            pltpu.VMEM((TB, NS * SLOT), jnp.bfloat16),
            pltpu.VMEM((TB, T * 2 * H), jnp.bfloat16),
            pltpu.VMEM((TB, OUTPAD), jnp.float32),
        ],
        compiler_params=pltpu.CompilerParams(
            dimension_semantics=("parallel",),
            vmem_limit_bytes=60 * 1024 * 1024,
        ),
    )(xr, p, ones_row, w, wfcb, bfct)

    return out[:B].reshape(B, T, I)


def kernel(x, wih0, whh0, b0, wih1, whh1, b1, wfc, bfc):
    return _forward(x, wih0, whh0, b0, wih1, whh1, b1, wfc, bfc)


# speedup vs baseline: 1.0135x; 1.0135x over previous
import functools

import jax
import jax.numpy as jnp
from jax import lax
from jax.experimental import pallas as pl
from jax.experimental.pallas import tpu as pltpu

I_SIZE = 3
H = 64
G = 8 * H
SLOT = 2 * H
OUTPAD = 128


def _gate_cols(w, layer):
    i, f, g, o = jnp.split(w, 4, axis=1)
    z = jnp.zeros_like(i)
    pairs = ((i, z), (f, z), (o, z), (g, z)) if layer == 0 else \
            ((z, i), (z, f), (z, o), (z, g))
    return jnp.concatenate([blk for pair in pairs for blk in pair], axis=1)


def _lstm_kernel(xr_ref, p_ref, ones_ref, w_ref, wfcb_ref, bfct_ref,
                 out_ref, xu, hseq, yacc, *, T, TB):
    NS = T + 1
    xr = xr_ref[...]

    def _write_xu(lo, hi):
        xu[:, lo * SLOT:hi * SLOT] = (
            jnp.dot(xr, p_ref[:, lo * SLOT:hi * SLOT],
                    preferred_element_type=jnp.float32)
            + ones_ref[:, lo * SLOT:hi * SLOT]).astype(jnp.bfloat16)

    w = w_ref[...]
    lane = lax.broadcasted_iota(jnp.int32, (TB, 2 * H), 1)
    l0_mask = lane < H

    h = jnp.zeros((TB, 2 * H), jnp.bfloat16)
    c = jnp.zeros((TB, 2 * H), jnp.float32)

    def _fc_chunk(k0, k1, first):
        part = jnp.dot(hseq[:, k0 * 2 * H:k1 * 2 * H],
                       wfcb_ref[k0 * 2 * H:k1 * 2 * H, :],
                       preferred_element_type=jnp.float32)
        yacc[...] = part if first else yacc[...] + part

    CH = 4
    fc_chunks = [(k, min(k + CH, T)) for k in range(0, T, CH)]
    fc_at = {k1 + 1: i for i, (k0, k1) in enumerate(fc_chunks) if k1 + 1 < NS}

    _write_xu(0, min(CH, NS))
    for s in range(NS):
        if s % CH == 1 and s + CH - 1 < NS:
            _write_xu(s + CH - 1, min(s + 2 * CH - 1, NS))
        if s in fc_at:
            k0, k1 = fc_chunks[fc_at[s]]
            _fc_chunk(k0, k1, first=(fc_at[s] == 0))
        hx = jnp.concatenate([h, xu[:, s * SLOT:(s + 1) * SLOT]], axis=1)
        z = jnp.dot(hx, w, preferred_element_type=jnp.float32)
        t = jnp.tanh(z)
        ti = t[:, :2 * H]
        tf = t[:, 2 * H:4 * H]
        to = t[:, 4 * H:6 * H]
        tg = t[:, 6 * H:]
        c = 0.5 * ((tf * c + c) + (ti * tg + tg))
        tc = jnp.tanh(c)
        hf = 0.5 * (to * tc + tc)
        if s == 0:
            hf = jnp.where(l0_mask, hf, 0.0)
            c = jnp.where(l0_mask, c, 0.0)
            h = hf.astype(jnp.bfloat16)
        else:
            h = hf.astype(jnp.bfloat16)
            hseq[:, (s - 1) * 2 * H:s * 2 * H] = h

    done = set(fc_at.values())
    for i, (k0, k1) in enumerate(fc_chunks):
        if i not in done:
            _fc_chunk(k0, k1, first=(i == 0))
    y = yacc[...] + bfct_ref[...]
    out_ref[...] = y[:, :T * I_SIZE].astype(out_ref.dtype)


@jax.jit
def _forward(x, wih0, whh0, b0, wih1, whh1, b1, wfc, bfc):
    B, T, I = x.shape
    TB = 768
    if B < TB:
        TB = max(8, -(-B // 8) * 8)
    Bpad = -(-B // TB) * TB
    nb = Bpad // TB
    xr = x.astype(jnp.bfloat16).reshape(B, T * I)
    if Bpad != B:
        xr = jnp.pad(xr, ((0, Bpad - B), (0, 0)))

    bb = _gate_cols(b0, 0) + _gate_cols(b1, 1)
    wx = _gate_cols(wih0, 0)
    wh = jnp.concatenate(
        [_gate_cols(whh0, 0) + _gate_cols(wih1, 1),
         _gate_cols(whh1, 1)], axis=0)

    wxe = jnp.zeros((SLOT, G), jnp.float32)
    wxe = wxe.at[:I, :].set(wx).at[I, :].set(bb[0])
    w = jnp.concatenate([wh, wxe], axis=0)
    w = w.at[:, :6 * H].multiply(0.5).astype(jnp.bfloat16)

    NS = T + 1
    p = jnp.zeros((T * I, NS * SLOT), jnp.float32)
    for t in range(T):
        p = p.at[t * I:(t + 1) * I, t * SLOT:t * SLOT + I].set(jnp.eye(I))
    p = p.astype(jnp.bfloat16)
    ones_row = jnp.zeros((1, NS * SLOT), jnp.float32)
    ones_row = ones_row.at[0, jnp.arange(NS) * SLOT + I].set(1.0)

    wfcb = jnp.zeros((T * 2 * H, OUTPAD), jnp.float32)
    for t in range(T):
        wfcb = wfcb.at[t * 2 * H + H:(t + 1) * 2 * H, t * I:(t + 1) * I].set(wfc)
    wfcb = wfcb.astype(jnp.bfloat16)
    bfct = jnp.zeros((1, OUTPAD), jnp.float32).at[:, :T * I].set(
        jnp.tile(bfc, (1, T)))

    out = pl.pallas_call(
        functools.partial(_lstm_kernel, T=T, TB=TB),
        out_shape=jax.ShapeDtypeStruct((Bpad, T * I), jnp.float32),
        grid=(nb,),
        in_specs=[
            pl.BlockSpec((TB, T * I), lambda i: (i, 0)),
            pl.BlockSpec((T * I, NS * SLOT), lambda i: (0, 0)),
            pl.BlockSpec((1, NS * SLOT), lambda i: (0, 0)),
            pl.BlockSpec((2 * H + SLOT, G), lambda i: (0, 0)),
            pl.BlockSpec((T * 2 * H, OUTPAD), lambda i: (0, 0)),
            pl.BlockSpec((1, OUTPAD), lambda i: (0, 0)),
        ],
        out_specs=pl.BlockSpec((TB, T * I), lambda i: (i, 0)),
        scratch_shapes=[
            pltpu.VMEM((TB, NS * SLOT), jnp.bfloat16),
            pltpu.VMEM((TB, T * 2 * H), jnp.bfloat16),
            pltpu.VMEM((TB, OUTPAD), jnp.float32),
        ],
        compiler_params=pltpu.CompilerParams(
            dimension_semantics=("parallel",),
            vmem_limit_bytes=60 * 1024 * 1024,
        ),
    )(xr, p, ones_row, w, wfcb, bfct)

    return out[:B].reshape(B, T, I)


def kernel(x, wih0, whh0, b0, wih1, whh1, b1, wfc, bfc):
    return _forward(x, wih0, whh0, b0, wih1, whh1, b1, wfc, bfc)
